# baseline (speedup 1.0000x reference)
import jax
import jax.numpy as jnp
from jax import lax
from jax.experimental import pallas as pl
from jax.experimental.pallas import tpu as pltpu
from jax.experimental.pallas import tpu_sc as plsc

Q = 64
D = 256
E = 4096
TOPK = 10
NC = 2
NS = 16
NW = NC * NS
L = 16
EPW = E // NW
QC = Q // L
RB = 2
DC = D // L


def _sc_body(qt_hbm, e_hbm, inter_hbm, qt_v, e_v, jv_v):
    c = lax.axis_index("c")
    s = lax.axis_index("s")
    wid = s * NC + c
    pltpu.sync_copy(qt_hbm, qt_v)
    pltpu.sync_copy(e_hbm.at[pl.ds(wid * EPW, EPW)], e_v)

    def blk_body(blk, _):
        rb = blk * RB

        def dc_body(dc, carry):
            accs = list(carry)
            d0 = dc * L
            evs = [e_v[rb + r, pl.ds(d0, L)] for r in range(RB)]
            for j in range(L):
                qvs = [qt_v[d0 + j, pl.ds(qc * L, L)] for qc in range(QC)]
                for r in range(RB):
                    eb = lax.broadcast(evs[r][j], (L,))
                    for qc in range(QC):
                        i = r * QC + qc
                        accs[i] = accs[i] + jnp.minimum(qvs[qc], eb)
            return tuple(accs)

        carry = lax.fori_loop(
            0, DC, dc_body,
            tuple(jnp.zeros((L,), jnp.float32) for _ in range(RB * QC)),
        )
        for r in range(RB):
            for qc in range(QC):
                jv_v[rb + r, pl.ds(qc * L, L)] = carry[r * QC + qc]
        return 0

    lax.fori_loop(0, EPW // RB, blk_body, 0)

    pltpu.sync_copy(jv_v, inter_hbm.at[pl.ds(wid * EPW, EPW), :])


def _tc_topk_body(i_ref, e_ref, qt_ref, top_ref, bot_ref):
    big = jnp.int32(2**30)
    neg = jnp.float32(-3e38)
    ent_iota = lax.broadcasted_iota(jnp.int32, (E, Q), 0)

    inter = i_ref[...]
    se = jnp.sum(e_ref[...], axis=1, keepdims=True)
    sq = jnp.sum(qt_ref[...], axis=0, keepdims=True)
    scores = inter / (sq + se - inter)

    def select10(cur, out_ref):
        for j in range(TOPK):
            m = jnp.max(cur, axis=0, keepdims=True)
            hit = cur == m
            ent = jnp.min(jnp.where(hit, ent_iota, big), axis=0,
                          keepdims=True)
            out_ref[j, :] = ent[0]
            cur = jnp.where(ent == ent_iota, neg, cur)

    select10(scores, top_ref)
    select10(-scores, bot_ref)


def kernel(query, enity_info, k):
    qt = query.reshape(Q, D).T
    e2 = enity_info.reshape(E, D)

    sc = pl.kernel(
        _sc_body,
        out_type=[jax.ShapeDtypeStruct((E, Q), jnp.float32)],
        mesh=plsc.VectorSubcoreMesh(
            core_axis_name="c", subcore_axis_name="s",
            num_cores=NC, num_subcores=NS,
        ),
        scratch_types=[
            pltpu.VMEM((D, Q), jnp.float32),
            pltpu.VMEM((EPW, D), jnp.float32),
            pltpu.VMEM((EPW, Q), jnp.float32),
        ],
    )
    inter, = sc(qt, e2)

    top, bot = pl.pallas_call(
        _tc_topk_body,
        out_shape=[
            jax.ShapeDtypeStruct((TOPK, Q), jnp.int32),
            jax.ShapeDtypeStruct((TOPK, Q), jnp.int32),
        ],
    )(inter, e2, qt)

    kd = jnp.asarray(k - TOPK, jnp.int32)
    return top.T + kd, bot.T + kd


# speedup vs baseline: 1.4464x; 1.4464x over previous
import jax
import jax.numpy as jnp
from jax import lax
from jax.experimental import pallas as pl
from jax.experimental.pallas import tpu as pltpu
from jax.experimental.pallas import tpu_sc as plsc

Q = 64
D = 256
E = 4096
TOPK = 10
NC = 2
NS = 16
NW = NC * NS
L = 16
EPW = E // NW
G = EPW // L
QB = 2
DC = D // L


def _sc_body(q_hbm, et_hbm, scores_hbm, q_v, et_v, jv_v):
    c = lax.axis_index("c")
    s = lax.axis_index("s")
    wid = s * NC + c
    pltpu.sync_copy(q_hbm, q_v)
    pltpu.sync_copy(et_hbm.at[wid], et_v)

    def se_body(d, accs):
        return tuple(accs[g] + et_v[d, pl.ds(g * L, L)] for g in range(G))

    se = lax.fori_loop(
        0, D, se_body, tuple(jnp.zeros((L,), jnp.float32) for _ in range(G))
    )

    def q_body(qp, _):
        qi = qp * QB

        def dc_body(dc, carry):
            accs = list(carry[:QB * G])
            sqs = list(carry[QB * G:])
            d0 = dc * L
            for j in range(L):
                qvs = [q_v[qi + b, pl.ds(d0, L)] for b in range(QB)]
                for g in range(G):
                    ev = et_v[d0 + j, pl.ds(g * L, L)]
                    for b in range(QB):
                        qb = lax.broadcast(qvs[b][j], (L,))
                        accs[b * G + g] = accs[b * G + g] + jnp.minimum(ev, qb)
                for b in range(QB):
                    sqs[b] = sqs[b] + lax.broadcast(qvs[b][j], (L,))
            return tuple(accs) + tuple(sqs)

        carry = lax.fori_loop(
            0, DC, dc_body,
            tuple(jnp.zeros((L,), jnp.float32) for _ in range(QB * G + QB)),
        )
        accs, sqs = carry[:QB * G], carry[QB * G:]
        for b in range(QB):
            for g in range(G):
                acc = accs[b * G + g]
                jv_v[qi + b, pl.ds(g * L, L)] = acc / (sqs[b] + se[g] - acc)
        return 0

    lax.fori_loop(0, Q // QB, q_body, 0)

    pltpu.sync_copy(jv_v, scores_hbm.at[:, pl.ds(wid * EPW, EPW)])


def _tc_topk_body(s_ref, top_ref, bot_ref):
    big = jnp.int32(2**30)
    neg = jnp.float32(-3e38)
    ent_iota = lax.broadcasted_iota(jnp.int32, (Q, E), 1)

    def select10(cur):
        outs = []
        for _ in range(TOPK):
            m = jnp.max(cur, axis=1, keepdims=True)
            hit = cur == m
            ent = jnp.min(jnp.where(hit, ent_iota, big), axis=1,
                          keepdims=True)
            outs.append(ent)
            cur = jnp.where(ent == ent_iota, neg, cur)
        return jnp.concatenate(outs, axis=1)

    scores = s_ref[...]
    top_ref[...] = select10(scores)
    bot_ref[...] = select10(-scores)


def kernel(query, enity_info, k):
    q2 = query.reshape(Q, D)
    eb = enity_info.reshape(NW, EPW, D).transpose(0, 2, 1)

    sc = pl.kernel(
        _sc_body,
        out_type=[jax.ShapeDtypeStruct((Q, E), jnp.float32)],
        mesh=plsc.VectorSubcoreMesh(
            core_axis_name="c", subcore_axis_name="s",
            num_cores=NC, num_subcores=NS,
        ),
        scratch_types=[
            pltpu.VMEM((Q, D), jnp.float32),
            pltpu.VMEM((D, EPW), jnp.float32),
            pltpu.VMEM((Q, EPW), jnp.float32),
        ],
    )
    scores, = sc(q2, eb)

    top, bot = pl.pallas_call(
        _tc_topk_body,
        out_shape=[
            jax.ShapeDtypeStruct((Q, TOPK), jnp.int32),
            jax.ShapeDtypeStruct((Q, TOPK), jnp.int32),
        ],
    )(scores)

    kd = jnp.asarray(k - TOPK, jnp.int32)
    return top + kd, bot + kd


# speedup vs baseline: 1.8659x; 1.2901x over previous
import jax
import jax.numpy as jnp
from jax import lax
from jax.experimental import pallas as pl
from jax.experimental.pallas import tpu as pltpu
from jax.experimental.pallas import tpu_sc as plsc

Q = 64
D = 256
E = 4096
TOPK = 10
NC = 2
NS = 16
NW = NC * NS
L = 16
EPW = E // NW
G = EPW // L
QB = 2
DC = D // L


def _sc_body(q_hbm, et_hbm, scores_hbm, q_v, et_v, jv_v):
    c = lax.axis_index("c")
    s = lax.axis_index("s")
    wid = s * NC + c
    pltpu.sync_copy(q_hbm, q_v)
    pltpu.sync_copy(et_hbm.at[wid], et_v)

    def se_body(d, accs):
        return tuple(accs[g] + et_v[d, pl.ds(g * L, L)] for g in range(G))

    se = lax.fori_loop(
        0, D, se_body, tuple(jnp.zeros((L,), jnp.float32) for _ in range(G))
    )

    GH = G // 2
    for h in range(2):
        g0 = h * GH

        def q_body(qp, _, g0=g0):
            qi = qp * QB

            def dc_body(dc, carry):
                accs = list(carry[:QB * GH])
                sqs = list(carry[QB * GH:])
                d0 = dc * L
                for j in range(L):
                    qvs = [q_v[qi + b, pl.ds(d0, L)] for b in range(QB)]
                    qbs = [lax.broadcast(qvs[b][j], (L,)) for b in range(QB)]
                    for g in range(GH):
                        ev = et_v[d0 + j, pl.ds((g0 + g) * L, L)]
                        for b in range(QB):
                            accs[b * GH + g] = accs[b * GH + g] + jnp.minimum(
                                ev, qbs[b]
                            )
                    for b in range(QB):
                        sqs[b] = sqs[b] + qbs[b]
                return tuple(accs) + tuple(sqs)

            carry = lax.fori_loop(
                0, DC, dc_body,
                tuple(jnp.zeros((L,), jnp.float32)
                      for _ in range(QB * GH + QB)),
            )
            accs, sqs = carry[:QB * GH], carry[QB * GH:]
            for b in range(QB):
                for g in range(GH):
                    acc = accs[b * GH + g]
                    jv_v[qi + b, pl.ds((g0 + g) * L, L)] = acc / (
                        sqs[b] + se[g0 + g] - acc
                    )
            return 0

        lax.fori_loop(0, Q // QB, q_body, 0)

    pltpu.sync_copy(jv_v, scores_hbm.at[:, pl.ds(wid * EPW, EPW)])


def _tc_topk_body(s_ref, top_ref, bot_ref):
    big = jnp.int32(2**30)
    neg = jnp.float32(-3e38)
    ent_iota = lax.broadcasted_iota(jnp.int32, (Q, E), 1)

    def select10(cur):
        outs = []
        for _ in range(TOPK):
            m = jnp.max(cur, axis=1, keepdims=True)
            hit = cur == m
            ent = jnp.min(jnp.where(hit, ent_iota, big), axis=1,
                          keepdims=True)
            outs.append(ent)
            cur = jnp.where(ent == ent_iota, neg, cur)
        return jnp.concatenate(outs, axis=1)

    scores = s_ref[...]
    top_ref[...] = select10(scores)
    bot_ref[...] = select10(-scores)


def kernel(query, enity_info, k):
    q2 = query.reshape(Q, D)
    eb = enity_info.reshape(NW, EPW, D).transpose(0, 2, 1)

    sc = pl.kernel(
        _sc_body,
        out_type=[jax.ShapeDtypeStruct((Q, E), jnp.float32)],
        mesh=plsc.VectorSubcoreMesh(
            core_axis_name="c", subcore_axis_name="s",
            num_cores=NC, num_subcores=NS,
        ),
        scratch_types=[
            pltpu.VMEM((Q, D), jnp.float32),
            pltpu.VMEM((D, EPW), jnp.float32),
            pltpu.VMEM((Q, EPW), jnp.float32),
        ],
    )
    scores, = sc(q2, eb)

    top, bot = pl.pallas_call(
        _tc_topk_body,
        out_shape=[
            jax.ShapeDtypeStruct((Q, TOPK), jnp.int32),
            jax.ShapeDtypeStruct((Q, TOPK), jnp.int32),
        ],
    )(scores)

    kd = jnp.asarray(k - TOPK, jnp.int32)
    return top + kd, bot + kd


# speedup vs baseline: 2.7335x; 1.4650x over previous
import jax
import jax.numpy as jnp
from jax import lax
from jax.experimental import pallas as pl
from jax.experimental.pallas import tpu as pltpu
from jax.experimental.pallas import tpu_sc as plsc

Q = 64
D = 256
E = 4096
TOPK = 10
NC = 2
NS = 16
NW = NC * NS
L = 16
ES = 2048
ET = E - ES
NP = NW // 2
EPW = ES // NP
G = EPW // L
QH = Q // 2
QB = 2
DC = D // L
TW = 256


def _sc_body(q_hbm, et_hbm, scores_hbm, q_v, et_v, jv_v):
    c = lax.axis_index("c")
    s = lax.axis_index("s")
    wid = s * NC + c
    pair = wid // 2
    qh = wid % 2
    pltpu.sync_copy(q_hbm.at[pl.ds(qh * QH, QH)], q_v)
    pltpu.sync_copy(et_hbm.at[pair], et_v)

    def se_body(d, accs):
        return tuple(accs[g] + et_v[d, pl.ds(g * L, L)] for g in range(G))

    se = lax.fori_loop(
        0, D, se_body, tuple(jnp.zeros((L,), jnp.float32) for _ in range(G))
    )

    GH = G // 2
    for h in range(2):
        g0 = h * GH

        def q_body(qp, _, g0=g0):
            qi = qp * QB

            def dc_body(dc, carry):
                accs = list(carry[:QB * GH])
                sqs = list(carry[QB * GH:])
                d0 = dc * L
                for j in range(L):
                    qvs = [q_v[qi + b, pl.ds(d0, L)] for b in range(QB)]
                    qbs = [lax.broadcast(qvs[b][j], (L,)) for b in range(QB)]
                    for g in range(GH):
                        ev = et_v[d0 + j, pl.ds((g0 + g) * L, L)]
                        for b in range(QB):
                            accs[b * GH + g] = accs[b * GH + g] + jnp.minimum(
                                ev, qbs[b]
                            )
                    for b in range(QB):
                        sqs[b] = sqs[b] + qbs[b]
                return tuple(accs) + tuple(sqs)

            carry = lax.fori_loop(
                0, DC, dc_body,
                tuple(jnp.zeros((L,), jnp.float32)
                      for _ in range(QB * GH + QB)),
            )
            accs, sqs = carry[:QB * GH], carry[QB * GH:]
            for b in range(QB):
                for g in range(GH):
                    acc = accs[b * GH + g]
                    jv_v[qi + b, pl.ds((g0 + g) * L, L)] = acc / (
                        sqs[b] + se[g0 + g] - acc
                    )
            return 0

        lax.fori_loop(0, QH // QB, q_body, 0)

    pltpu.sync_copy(
        jv_v, scores_hbm.at[pl.ds(qh * QH, QH), pl.ds(pair * EPW, EPW)]
    )


def _tc_minsum_body(qc_ref, et_ref, out_ref):
    for t in range(ET // TW):
        def db_body(db, carry):
            acc, seacc, sqacc = carry
            d0 = db * 8
            qc8 = qc_ref[pl.ds(d0, 8)]
            er8 = et_ref[pl.ds(d0, 8), pl.ds(t * TW, TW)]
            for j in range(8):
                qcol = qc8[j]
                erow = er8[j:j + 1, :]
                acc = acc + jnp.minimum(qcol, erow)
                seacc = seacc + erow
                sqacc = sqacc + qcol
            return acc, seacc, sqacc

        acc, seacc, sqacc = lax.fori_loop(
            0, D // 8, db_body,
            (jnp.zeros((Q, TW), jnp.float32),
             jnp.zeros((1, TW), jnp.float32),
             jnp.zeros((Q, 1), jnp.float32)),
        )
        out_ref[:, pl.ds(t * TW, TW)] = acc / (sqacc + seacc - acc)


def _tc_topk_body(s1_ref, s2_ref, top_ref, bot_ref):
    big = jnp.int32(2**30)
    neg = jnp.float32(-3e38)
    ent_iota = lax.broadcasted_iota(jnp.int32, (Q, E), 1)

    def select10(cur):
        outs = []
        for _ in range(TOPK):
            m = jnp.max(cur, axis=1, keepdims=True)
            hit = cur == m
            ent = jnp.min(jnp.where(hit, ent_iota, big), axis=1,
                          keepdims=True)
            outs.append(ent)
            cur = jnp.where(ent == ent_iota, neg, cur)
        return jnp.concatenate(outs, axis=1)

    scores = jnp.concatenate([s1_ref[...], s2_ref[...]], axis=1)
    top_ref[...] = select10(scores)
    bot_ref[...] = select10(-scores)


def kernel(query, enity_info, k):
    q2 = query.reshape(Q, D)
    e2 = enity_info.reshape(E, D)
    eb_sc = e2[:ES].reshape(NP, EPW, D).transpose(0, 2, 1)
    et_tc = e2[ES:].T

    sc = pl.kernel(
        _sc_body,
        out_type=[jax.ShapeDtypeStruct((Q, ES), jnp.float32)],
        mesh=plsc.VectorSubcoreMesh(
            core_axis_name="c", subcore_axis_name="s",
            num_cores=NC, num_subcores=NS,
        ),
        scratch_types=[
            pltpu.VMEM((QH, D), jnp.float32),
            pltpu.VMEM((D, EPW), jnp.float32),
            pltpu.VMEM((QH, EPW), jnp.float32),
        ],
    )
    scores_sc, = sc(q2, eb_sc)

    qc3 = q2.T.reshape(D, Q, 1)
    scores_tc = pl.pallas_call(
        _tc_minsum_body,
        out_shape=jax.ShapeDtypeStruct((Q, ET), jnp.float32),
    )(qc3, et_tc)

    top, bot = pl.pallas_call(
        _tc_topk_body,
        out_shape=[
            jax.ShapeDtypeStruct((Q, TOPK), jnp.int32),
            jax.ShapeDtypeStruct((Q, TOPK), jnp.int32),
        ],
    )(scores_sc, scores_tc)

    kd = jnp.asarray(k - TOPK, jnp.int32)
    return top + kd, bot + kd
